# Initial kernel scaffold; baseline (speedup 1.0000x reference)
import jax
import jax.numpy as jnp
from jax.experimental import pallas as pl


def kernel(h, batch_idx, W_gate, b_gate, W_t, b_t, W_out, b_out):
    raise NotImplementedError("write your pallas kernel here")


# speedup vs baseline: 1.8698x; 1.8698x over previous
import functools

import jax
import jax.numpy as jnp
from jax import lax
from jax.experimental import pallas as pl
from jax.experimental.pallas import tpu as pltpu
from jax.experimental.pallas import tpu_sc as plsc

N = 320000
D = 128
NSEG = 512
ACC_W = D + 16
NC, NS, L = 2, 16, 16
NW = NC * NS
ROWS_PER_W = N // NW
RBLK = 80
NBLK = ROWS_PER_W // RBLK


def _sc_segsum(h, idx, wg, bg16, zacc):
  mesh = plsc.VectorSubcoreMesh(
      core_axis_name="c", subcore_axis_name="s", num_cores=NC,
      num_subcores=NS)

  @functools.partial(
      pl.kernel,
      out_type=jax.ShapeDtypeStruct((NW, NSEG, ACC_W), jnp.float32),
      mesh=mesh,
      scratch_types=[
          pltpu.VMEM((NSEG, ACC_W), jnp.float32),
          pltpu.VMEM((RBLK, D), jnp.float32),
          pltpu.VMEM((RBLK,), jnp.int32),
          pltpu.VMEM((D,), jnp.float32),
          pltpu.VMEM((L,), jnp.float32),
      ],
      compiler_params=pltpu.CompilerParams(needs_layout_passes=False,
                                           use_tc_tiling_on_sc=False),
  )
  def k(h_hbm, idx_hbm, wg_hbm, bg_hbm, z_hbm, out_hbm,
        acc_v, h_v, idx_v, wg_v, bg_v):
    wid = lax.axis_index("c") * NS + lax.axis_index("s")
    base0 = wid * ROWS_PER_W

    pltpu.sync_copy(z_hbm, acc_v)
    pltpu.sync_copy(wg_hbm, wg_v)
    pltpu.sync_copy(bg_hbm, bg_v)

    w_regs = [wg_v[pl.ds(16 * k2, 16)] for k2 in range(D // L)]
    bgv = bg_v[...]

    def block_body(b, _):
      base = base0 + b * RBLK
      pltpu.sync_copy(h_hbm.at[pl.ds(base, RBLK), :], h_v)
      pltpu.sync_copy(idx_hbm.at[pl.ds(base, RBLK)], idx_v)

      def group_body(g, _):
        iv = idx_v[pl.ds(g * L, L)]
        for j in range(L):
          r = g * L + j
          seg = iv[j]
          rows = [h_v[r, pl.ds(16 * k2, 16)] for k2 in range(D // L)]
          acc = rows[0] * w_regs[0]
          for k2 in range(1, D // L):
            acc = acc + rows[k2] * w_regs[k2]
          tot = jnp.sum(acc)
          x = tot + bgv
          gate = 1.0 / (1.0 + jnp.exp(-x))
          for k2 in range(D // L):
            plsc.addupdate(acc_v.at[seg, pl.ds(16 * k2, 16)],
                           gate * rows[k2])
          plsc.addupdate(acc_v.at[seg, pl.ds(D, 16)], gate)
        return 0

      lax.fori_loop(0, RBLK // L, group_body, 0)
      return 0

    lax.fori_loop(0, NBLK, block_body, 0)
    pltpu.sync_copy(acc_v, out_hbm.at[wid])

  return k(h, idx, wg, bg16, zacc)


def _tc_finish_body(p_ref, wt_ref, bt_ref, wo_ref, bo_ref, o_ref):
  a = jnp.sum(p_ref[...], axis=0)
  g = a[:, :D]
  c = a[:, D:D + 1]
  pooled = jnp.dot(g, wt_ref[...], preferred_element_type=jnp.float32)
  pooled = pooled + c * bt_ref[...][None, :]
  out = jnp.dot(pooled, wo_ref[...], preferred_element_type=jnp.float32)
  o_ref[...] = out + bo_ref[...][None, :]


def kernel(h, batch_idx, W_gate, b_gate, W_t, b_t, W_out, b_out):
  idx = batch_idx.astype(jnp.int32)
  wg = W_gate.reshape(D)
  bg16 = jnp.broadcast_to(b_gate.reshape(1), (L,)).astype(jnp.float32)
  zacc = jnp.zeros((NSEG, ACC_W), jnp.float32)

  partials = _sc_segsum(h, idx, wg, bg16, zacc)

  return pl.pallas_call(
      _tc_finish_body,
      out_shape=jax.ShapeDtypeStruct((NSEG, D), jnp.float32),
  )(partials, W_t, b_t, W_out, b_out)


# speedup vs baseline: 3.9398x; 2.1071x over previous
import functools

import jax
import jax.numpy as jnp
from jax import lax
from jax.experimental import pallas as pl
from jax.experimental.pallas import tpu as pltpu
from jax.experimental.pallas import tpu_sc as plsc

N = 320000
D = 128
NSEG = 512
ACC_W = D + 16
NC, NS, L = 2, 16, 16
NW = NC * NS
ROWS_PER_W = N // NW
RBLK = 80
NBLK = ROWS_PER_W // RBLK


def _sc_segsum(h, idx, wg, bg16, zacc):
  mesh = plsc.VectorSubcoreMesh(
      core_axis_name="c", subcore_axis_name="s", num_cores=NC,
      num_subcores=NS)

  @functools.partial(
      pl.kernel,
      out_type=jax.ShapeDtypeStruct((NW, NSEG, ACC_W), jnp.float32),
      mesh=mesh,
      scratch_types=[
          pltpu.VMEM((NSEG, ACC_W), jnp.float32),
          pltpu.VMEM((RBLK, D), jnp.float32),
          pltpu.VMEM((RBLK, D), jnp.float32),
          pltpu.VMEM((RBLK,), jnp.int32),
          pltpu.VMEM((RBLK,), jnp.int32),
          pltpu.VMEM((D,), jnp.float32),
          pltpu.VMEM((L,), jnp.float32),
          pltpu.SemaphoreType.DMA,
          pltpu.SemaphoreType.DMA,
      ],
      compiler_params=pltpu.CompilerParams(needs_layout_passes=False,
                                           use_tc_tiling_on_sc=False),
  )
  def k(h_hbm, idx_hbm, wg_hbm, bg_hbm, z_hbm, out_hbm,
        acc_v, h_v0, h_v1, idx_v0, idx_v1, wg_v, bg_v, sem0, sem1):
    wid = lax.axis_index("c") * NS + lax.axis_index("s")
    base0 = wid * ROWS_PER_W
    bufs = ((h_v0, idx_v0, sem0), (h_v1, idx_v1, sem1))

    def start_copies(b, hbuf, ibuf, sem):
      base = base0 + b * RBLK
      pltpu.make_async_copy(h_hbm.at[pl.ds(base, RBLK), :], hbuf, sem).start()
      pltpu.make_async_copy(idx_hbm.at[pl.ds(base, RBLK)], ibuf, sem).start()

    def wait_copies(b, hbuf, ibuf, sem):
      base = base0 + b * RBLK
      pltpu.make_async_copy(h_hbm.at[pl.ds(base, RBLK), :], hbuf, sem).wait()
      pltpu.make_async_copy(idx_hbm.at[pl.ds(base, RBLK)], ibuf, sem).wait()

    pltpu.sync_copy(z_hbm, acc_v)
    pltpu.sync_copy(wg_hbm, wg_v)
    pltpu.sync_copy(bg_hbm, bg_v)

    w_regs = [wg_v[pl.ds(16 * k2, 16)] for k2 in range(D // L)]
    bgv = bg_v[...]
    NK = D // L

    def process_block(h_v, idx_v):
      def group_body(g, _):
        iv = idx_v[pl.ds(g * L, L)]
        gb = []
        for j in range(L):
          r = g * L + j
          acc = h_v[r, pl.ds(0, 16)] * w_regs[0]
          for k2 in range(1, NK):
            acc = acc + h_v[r, pl.ds(16 * k2, 16)] * w_regs[k2]
          x = jnp.broadcast_to(jnp.sum(acc), (L,)) + bgv
          gb.append(1.0 / (1.0 + jnp.exp(-x)))
        gsum = gb[0]
        for j in range(1, L):
          gsum = gsum + gb[j]
        seg0 = iv[0]
        seg15 = iv[L - 1]

        def fast(_):
          for k2 in range(NK):
            a = gb[0] * h_v[g * L, pl.ds(16 * k2, 16)]
            for j in range(1, L):
              a = a + gb[j] * h_v[g * L + j, pl.ds(16 * k2, 16)]
            plsc.addupdate(acc_v.at[seg0, pl.ds(16 * k2, 16)], a)
          plsc.addupdate(acc_v.at[seg0, pl.ds(D, 16)], gsum)
          return 0

        def slow(_):
          for j in range(L):
            seg = iv[j]
            for k2 in range(NK):
              plsc.addupdate(acc_v.at[seg, pl.ds(16 * k2, 16)],
                             gb[j] * h_v[g * L + j, pl.ds(16 * k2, 16)])
            plsc.addupdate(acc_v.at[seg, pl.ds(D, 16)], gb[j])
          return 0

        lax.cond(seg0 == seg15, fast, slow, 0)
        return 0

      lax.fori_loop(0, RBLK // L, group_body, 0)

    start_copies(0, *bufs[0])
    start_copies(1, *bufs[1])

    def pair_body(p, _):
      for par in range(2):
        b = 2 * p + par
        hbuf, ibuf, sem = bufs[par]

        @pl.when(b < NBLK)
        def _():
          wait_copies(b, hbuf, ibuf, sem)
          process_block(hbuf, ibuf)

          @pl.when(b + 2 < NBLK)
          def _():
            start_copies(b + 2, hbuf, ibuf, sem)
      return 0

    lax.fori_loop(0, (NBLK + 1) // 2, pair_body, 0)
    pltpu.sync_copy(acc_v, out_hbm.at[wid])

  return k(h, idx, wg, bg16, zacc)


def _tc_finish_body(p_ref, wt_ref, bt_ref, wo_ref, bo_ref, o_ref):
  a = jnp.sum(p_ref[...], axis=0)
  g = a[:, :D]
  c = a[:, D:D + 1]
  pooled = jnp.dot(g, wt_ref[...], preferred_element_type=jnp.float32)
  pooled = pooled + c * bt_ref[...][None, :]
  out = jnp.dot(pooled, wo_ref[...], preferred_element_type=jnp.float32)
  o_ref[...] = out + bo_ref[...][None, :]


def kernel(h, batch_idx, W_gate, b_gate, W_t, b_t, W_out, b_out):
  idx = batch_idx.astype(jnp.int32)
  wg = W_gate.reshape(D)
  bg16 = jnp.broadcast_to(b_gate.reshape(1), (L,)).astype(jnp.float32)
  zacc = jnp.zeros((NSEG, ACC_W), jnp.float32)

  partials = _sc_segsum(h, idx, wg, bg16, zacc)

  return pl.pallas_call(
      _tc_finish_body,
      out_shape=jax.ShapeDtypeStruct((NSEG, D), jnp.float32),
  )(partials, W_t, b_t, W_out, b_out)
